# baseline (speedup 1.0000x reference)
import functools

import jax
import jax.numpy as jnp
from jax.experimental import pallas as pl
from jax.experimental.pallas import tpu as pltpu
from jax.experimental.pallas import tpu_sc as plsc

_B, _C, _T, _H, _W = 2, 64, 8, 16, 16
_K = 8192
_BETA = 0.25
_NG = 16
_N = _B * _T * _H * _W
_TB = 256
_KC = 512
_THW = _T * _H * _W
_HW = _H * _W
_BC = _B * _C
_F32 = jnp.float32


def _vq_body(zT_ref, emb_ref, idx_ref, avg_ref, esq_scr, d_scr, colf_scr):
    i = pl.program_id(0)
    nch = _K // _KC
    z = jnp.transpose(zT_ref[...], (1, 0))
    zsq = jnp.sum(z * z, axis=1, keepdims=True)

    @pl.when(i == 0)
    def _init():
        for c in range(nch):
            e = emb_ref[pl.ds(c * _KC, _KC), :]
            esq_scr[:, c * _KC:(c + 1) * _KC] = jnp.transpose(
                jnp.sum(e * e, axis=1, keepdims=True), (1, 0))
        colf_scr[...] = jax.lax.broadcasted_iota(
            jnp.int32, (_TB, _K), 1).astype(_F32)
        avg_ref[...] = jnp.zeros_like(avg_ref)

    dm = None
    sm = None
    for c in range(nch):
        e = emb_ref[pl.ds(c * _KC, _KC), :]
        esq = esq_scr[:, c * _KC:(c + 1) * _KC]
        mm = jax.lax.dot_general(z, e, (((1,), (1,)), ((), ())),
                                 preferred_element_type=_F32)
        d = zsq + esq - 2.0 * mm
        d_scr[:, c * _KC:(c + 1) * _KC] = d
        r = 1.0 / d
        dm = d if c == 0 else jnp.minimum(dm, d)
        sm = r if c == 0 else sm + r
    dmin = jnp.min(dm, axis=1, keepdims=True)
    s = jnp.sum(sm, axis=1, keepdims=True)
    im = None
    for c in range(nch):
        d = d_scr[:, c * _KC:(c + 1) * _KC]
        cand = jnp.where(d == dmin, colf_scr[:, c * _KC:(c + 1) * _KC],
                         float(2 * _K))
        im = cand if c == 0 else jnp.minimum(im, cand)
    a = jnp.min(im, axis=1, keepdims=True)
    idx_ref[...] = a.astype(jnp.int32)
    sinvT = jnp.transpose(1.0 / s, (1, 0))
    for c in range(nch):
        r = 1.0 / d_scr[:, c * _KC:(c + 1) * _KC]
        avg_ref[:, c * _KC:(c + 1) * _KC] += jax.lax.dot_general(
            sinvT, r, (((1,), (0,)), ((), ())), preferred_element_type=_F32)


def _vq(zT, emb):
    return pl.pallas_call(
        _vq_body,
        grid=(_N // _TB,),
        in_specs=[pl.BlockSpec((_C, _HW), lambda i: (i // 8, i % 8)),
                  pl.BlockSpec((_K, _C), lambda i: (0, 0))],
        out_specs=[pl.BlockSpec((_TB, 1), lambda i: (i, 0)),
                   pl.BlockSpec((1, _K), lambda i: (0, 0))],
        out_shape=[jax.ShapeDtypeStruct((_N, 1), jnp.int32),
                   jax.ShapeDtypeStruct((1, _K), _F32)],
        scratch_shapes=[pltpu.VMEM((1, _K), _F32),
                        pltpu.VMEM((_TB, _K), _F32),
                        pltpu.VMEM((_TB, _K), _F32)],
    )(zT, emb)


def _attn_body(x_ref, gw_ref, M_ref, Wq_ref, Wk_ref, Wv_ref, Wp_ref, bias_ref,
               zt_ref, h_scr, q_scr, k_scr, v_scr):
    x = x_ref[...]
    rs = jnp.sum(x, axis=1, keepdims=True)
    rs2 = jnp.sum(x * x, axis=1, keepdims=True)
    M = M_ref[...]
    mu = jax.lax.dot_general(M, rs, (((1,), (0,)), ((), ())),
                             precision=jax.lax.Precision.HIGHEST,
                             preferred_element_type=_F32)
    ex2 = jax.lax.dot_general(M, rs2, (((1,), (0,)), ((), ())),
                              precision=jax.lax.Precision.HIGHEST,
                              preferred_element_type=_F32)
    var = ex2 - mu * mu
    gr = gw_ref[:, 0:1]
    br = gw_ref[:, 1:2]
    h_scr[...] = ((x - mu) / jnp.sqrt(var + 1e-6)) * gr + br
    bq = bias_ref[:, 0:1]
    bk = bias_ref[:, 1:2]
    bv = bias_ref[:, 2:3]
    bp = bias_ref[:, 3:4]
    for b in range(_B):
        hb = h_scr[pl.ds(b * _C, _C), :]
        q_scr[pl.ds(b * _C, _C), :] = jax.lax.dot_general(
            Wq_ref[...], hb, (((1,), (0,)), ((), ())),
            preferred_element_type=_F32) + bq
        k_scr[pl.ds(b * _C, _C), :] = jax.lax.dot_general(
            Wk_ref[...], hb, (((1,), (0,)), ((), ())),
            preferred_element_type=_F32) + bk
        v_scr[pl.ds(b * _C, _C), :] = jax.lax.dot_general(
            Wv_ref[...], hb, (((1,), (0,)), ((), ())),
            preferred_element_type=_F32) + bv
    scale = float(_C) ** (-0.5)
    bfr = lambda u: u.astype(jnp.bfloat16).astype(_F32)
    for b in range(_B):
        vb = bfr(v_scr[pl.ds(b * _C, _C), :])
        for i in range(_T):
            qi = bfr(q_scr[pl.ds(b * _C, _C), pl.ds(i * _HW, _HW)])
            rows = []
            for j in range(_T):
                kj = bfr(k_scr[pl.ds(b * _C, _C), pl.ds(j * _HW, _HW)])
                rows.append(jnp.sum(qi * kj, axis=0, keepdims=True) * scale)
            s = jnp.concatenate(rows, axis=0)
            p = bfr(jax.nn.softmax(s, axis=0))
            h2 = p[0:1, :] * vb[:, 0:_HW]
            for j in range(1, _T):
                h2 = h2 + p[j:j + 1, :] * vb[:, j * _HW:(j + 1) * _HW]
            h_scr[pl.ds(b * _C, _C), pl.ds(i * _HW, _HW)] = h2
    for b in range(_B):
        xb = x_ref[pl.ds(b * _C, _C), :]
        tdiff = jnp.concatenate(
            [jnp.zeros((_C, _HW), _F32),
             xb[:, _HW:_THW] - xb[:, 0:_THW - _HW]],
            axis=1)
        zt_ref[pl.ds(b * _C, _C), :] = jax.lax.dot_general(
            Wp_ref[...], h_scr[pl.ds(b * _C, _C), :], (((1,), (0,)), ((), ())),
            preferred_element_type=_F32) + bp + tdiff


def _attn(xT, gn_g, gn_b, Wq, bq, Wk, bk, Wv, bv, Wp, bp):
    gw = jnp.stack([jnp.tile(gn_g, (_B,)), jnp.tile(gn_b, (_B,))], axis=1)
    row = jax.lax.broadcasted_iota(jnp.int32, (_BC, _BC), 0)
    col = jax.lax.broadcasted_iota(jnp.int32, (_BC, _BC), 1)
    M = jnp.where(row // 4 == col // 4, 1.0 / (4 * _THW), 0.0).astype(_F32)
    bias = jnp.stack([bq, bk, bv, bp], axis=1)
    full = lambda a, b_: pl.BlockSpec((a, b_), lambda: (0, 0))
    return pl.pallas_call(
        _attn_body,
        grid=(),
        in_specs=[full(_BC, _THW), full(_BC, 2), full(_BC, _BC),
                  full(_C, _C), full(_C, _C), full(_C, _C), full(_C, _C),
                  full(_C, 4)],
        out_specs=full(_BC, _THW),
        out_shape=jax.ShapeDtypeStruct((_BC, _THW), _F32),
        scratch_shapes=[pltpu.VMEM((_BC, _THW), _F32),
                        pltpu.VMEM((_BC, _THW), _F32),
                        pltpu.VMEM((_BC, _THW), _F32),
                        pltpu.VMEM((_BC, _THW), _F32)],
    )(xT, gw, M, Wq, Wk, Wv, Wp, bias)


def _sc_gather_hist(emb_s, emb_t, idx_s, idx_t):
    info = plsc.get_sparse_core_info()
    nc, ns = info.num_cores, info.num_subcores
    nw = nc * ns
    bpw = _N // nw
    ksl = _K // ns
    mesh = plsc.VectorSubcoreMesh(core_axis_name="c", subcore_axis_name="s")

    @functools.partial(
        pl.kernel, mesh=mesh,
        out_type=[jax.ShapeDtypeStruct((_N, 2 * _C), _F32),
                  jax.ShapeDtypeStruct((_N, 2 * _C), _F32),
                  jax.ShapeDtypeStruct((nc * _K,), _F32),
                  jax.ShapeDtypeStruct((nc * _K,), _F32)],
        scratch_types=[pltpu.VMEM((bpw,), jnp.int32),
                       pltpu.VMEM((bpw, 2 * _C), _F32),
                       pltpu.VMEM((bpw,), _F32),
                       pltpu.VMEM((ksl,), _F32),
                       pltpu.VMEM_SHARED((_K,), _F32),
                       pltpu.VMEM_SHARED((_K,), _F32),
                       pltpu.SemaphoreType.DMA],
    )
    def k(embs_h, embt_h, idxs_h, idxt_h, qs_h, qt_h, cs_h, ct_h,
          idx_v, rows_v, ones_v, slice_v, hs_sh, ht_sh, sem):
        cid = jax.lax.axis_index("c")
        sid = jax.lax.axis_index("s")
        wid = sid * nc + cid
        base = wid * bpw

        def _fill(i, _):
            ones_v[pl.ds(i * 16, 16)] = jnp.ones((16,), _F32)
            return 0
        jax.lax.fori_loop(0, bpw // 16, _fill, 0)

        @pl.when(sid == 0)
        def _zero():
            def _z(i, _):
                slice_v[pl.ds(i * 16, 16)] = jnp.zeros((16,), _F32)
                return 0
            jax.lax.fori_loop(0, ksl // 16, _z, 0)
            for sh in (hs_sh, ht_sh):
                for j in range(ns):
                    pltpu.sync_copy(slice_v, sh.at[pl.ds(j * ksl, ksl)])
        plsc.subcore_barrier()

        for idxh, embh, qh, sh in ((idxs_h, embs_h, qs_h, hs_sh),
                                   (idxt_h, embt_h, qt_h, ht_sh)):
            pltpu.sync_copy(idxh.at[pl.ds(base, bpw)], idx_v)
            pltpu.async_copy(embh.at[idx_v], rows_v, sem).wait()
            pltpu.sync_copy(rows_v, qh.at[pl.ds(base, bpw)])
            pltpu.sync_copy(ones_v, sh.at[idx_v], add=True)
        plsc.subcore_barrier()

        for sh, ch in ((hs_sh, cs_h), (ht_sh, ct_h)):
            pltpu.sync_copy(sh.at[pl.ds(sid * ksl, ksl)], slice_v)
            pltpu.sync_copy(slice_v, ch.at[pl.ds(cid * _K + sid * ksl, ksl)])

    return k(emb_s, emb_t, idx_s, idx_t)


def _fin_body(zT_ref, qs_ref, qt_ref, avgs_ref, avgt_ref, cs_ref, ct_ref,
              zq_ref, loss_ref, l2_ref, ps_ref, pt_ref):
    acc = jnp.zeros((1, 1), _F32)
    for i in range(_N // _TB):
        rb, cb = i // 8, i % 8
        qs = qs_ref[pl.ds(i * _TB, _TB), 0:_C]
        qt = qt_ref[pl.ds(i * _TB, _TB), 0:_C]
        zq = qt + qs
        z64 = zT_ref[pl.ds(rb * _C, _C), pl.ds(cb * _HW, _HW)]
        zrows = jnp.transpose(z64, (1, 0))
        diff = zq - zrows
        acc = acc + jnp.sum(diff * diff).reshape(1, 1)
        zq_ref[pl.ds(rb * _C, _C), pl.ds(cb * _HW, _HW)] = jnp.transpose(
            zrows + diff, (1, 0))
    msq = acc * (1.0 / float(_N * _C))
    loss_ref[...] = msq + _BETA * msq

    u = jnp.full((1, _K), 1.0 / _K, _F32)
    pu = jax.nn.softmax(u, axis=1)
    lpu = jnp.log(pu)
    l2 = jnp.zeros((1, 1), _F32)
    for avg_ref in (avgs_ref, avgt_ref):
        avg = avg_ref[...] * (1.0 / float(_N))
        lsm = jax.nn.log_softmax(avg, axis=1)
        l2 = l2 + jnp.mean(pu * (lpu - lsm)).reshape(1, 1)
    l2_ref[...] = 0.1 * l2

    for c_ref, p_ref in ((cs_ref, ps_ref), (ct_ref, pt_ref)):
        cnt = c_ref[0:1, :] + c_ref[1:2, :]
        e = cnt * (1.0 / float(_N))
        p_ref[...] = jnp.exp(-jnp.sum(e * jnp.log(e + 1e-10))).reshape(1, 1)


def _finalize(zT, q_s, q_t, avg_s, avg_t, cnt_s, cnt_t):
    full = lambda a, b_: pl.BlockSpec((a, b_), lambda: (0, 0))
    one = jax.ShapeDtypeStruct((1, 1), _F32)
    return pl.pallas_call(
        _fin_body,
        grid=(),
        in_specs=[full(_BC, _THW), full(_N, 2 * _C), full(_N, 2 * _C),
                  full(1, _K), full(1, _K), full(2, _K), full(2, _K)],
        out_specs=[full(_BC, _THW), full(1, 1), full(1, 1), full(1, 1),
                   full(1, 1)],
        out_shape=[jax.ShapeDtypeStruct((_BC, _THW), _F32), one, one, one,
                   one],
    )(zT, q_s, q_t, avg_s, avg_t, cnt_s, cnt_t)


def kernel(z, emb_s, emb_t, gn_g, gn_b, Wq, bq, Wk, bk, Wv, bv, Wp, bp):
    b, c, t, h, w = z.shape
    zT = z.reshape(_BC, _THW)

    idx_s2, avg_s = _vq(zT, emb_s)
    idx_s = idx_s2.reshape(-1)
    idx_s_out = jnp.transpose(idx_s.reshape(t, b, h, w),
                              (1, 0, 2, 3)).reshape(b * t, h * w)

    zt_T = _attn(zT, gn_g, gn_b, Wq, bq, Wk, bk, Wv, bv, Wp, bp)
    idx_t2, avg_t = _vq(zt_T, emb_t)
    idx_t = idx_t2.reshape(-1)
    idx_t_out = jnp.transpose(idx_t.reshape(t, b, h, w),
                              (1, 0, 2, 3)).reshape(b * t, h * w)

    pad = jnp.zeros((_K, _C), _F32)
    q_s, q_t, cnt_s, cnt_t = _sc_gather_hist(
        jnp.concatenate([emb_s, pad], axis=1),
        jnp.concatenate([emb_t, pad], axis=1), idx_s, idx_t)

    zqT, loss, l2, perp_s, perp_t = _finalize(
        zT, q_s, q_t, avg_s, avg_t, cnt_s.reshape(2, _K), cnt_t.reshape(2, _K))

    return (zqT.reshape(z.shape), loss.reshape(()), l2.reshape(()),
            idx_s_out, idx_t_out, perp_s.reshape(()), perp_t.reshape(()))


# speedup vs baseline: 1.1096x; 1.1096x over previous
import functools

import jax
import jax.numpy as jnp
from jax.experimental import pallas as pl
from jax.experimental.pallas import tpu as pltpu
from jax.experimental.pallas import tpu_sc as plsc

_B, _C, _T, _H, _W = 2, 64, 8, 16, 16
_K = 8192
_BETA = 0.25
_NG = 16
_N = _B * _T * _H * _W
_TB = 256
_KC = 1024
_THW = _T * _H * _W
_HW = _H * _W
_BC = _B * _C
_F32 = jnp.float32


def _vq_body(zT_ref, embT_ref, idx_ref, avg_ref, esq_scr, r_scr):
    i = pl.program_id(0)
    nch = _K // _KC
    z = jnp.transpose(zT_ref[...], (1, 0))
    zsq = jnp.sum(z * z, axis=1, keepdims=True)

    @pl.when(i == 0)
    def _init():
        for c in range(nch):
            eT = embT_ref[:, c * _KC:(c + 1) * _KC]
            esq_scr[:, c * _KC:(c + 1) * _KC] = jnp.sum(
                eT * eT, axis=0, keepdims=True)
        avg_ref[...] = jnp.zeros_like(avg_ref)

    m = jnp.full((_TB, 1), jnp.inf, _F32)
    a = jnp.full((_TB, 1), 0.0, _F32)
    s = jnp.zeros((_TB, 1), _F32)
    for c in range(nch):
        eT = embT_ref[:, c * _KC:(c + 1) * _KC]
        esq = esq_scr[:, c * _KC:(c + 1) * _KC]
        mm = jax.lax.dot_general(z, eT, (((1,), (0,)), ((), ())),
                                 preferred_element_type=_F32)
        d = zsq + esq - 2.0 * mm
        r = 1.0 / d
        r_scr[:, c * _KC:(c + 1) * _KC] = r
        dmin = jnp.min(d, axis=1, keepdims=True)
        colf = jax.lax.broadcasted_iota(
            jnp.int32, (_TB, _KC), 1).astype(_F32) + float(c * _KC)
        idxc = jnp.min(jnp.where(d == dmin, colf, float(_K)),
                       axis=1, keepdims=True)
        upd = dmin < m
        a = jnp.where(upd, idxc, a)
        m = jnp.where(upd, dmin, m)
        s = s + jnp.sum(r, axis=1, keepdims=True)
    idx_ref[...] = a.astype(jnp.int32)
    sinvT = jnp.transpose(1.0 / s, (1, 0))
    for c in range(nch):
        r = r_scr[:, c * _KC:(c + 1) * _KC]
        avg_ref[:, c * _KC:(c + 1) * _KC] += jax.lax.dot_general(
            sinvT, r, (((1,), (0,)), ((), ())), preferred_element_type=_F32)


def _vq(zT, embT):
    return pl.pallas_call(
        _vq_body,
        grid=(_N // _TB,),
        in_specs=[pl.BlockSpec((_C, _HW), lambda i: (i // 8, i % 8)),
                  pl.BlockSpec((_C, _K), lambda i: (0, 0))],
        out_specs=[pl.BlockSpec((_TB, 1), lambda i: (i, 0)),
                   pl.BlockSpec((1, _K), lambda i: (0, 0))],
        out_shape=[jax.ShapeDtypeStruct((_N, 1), jnp.int32),
                   jax.ShapeDtypeStruct((1, _K), _F32)],
        scratch_shapes=[pltpu.VMEM((1, _K), _F32),
                        pltpu.VMEM((_TB, _K), _F32)],
    )(zT, embT)


def _attn_body(x_ref, gw_ref, M_ref, Wq_ref, Wk_ref, Wv_ref, Wp_ref, bias_ref,
               zt_ref, h_scr, q_scr, k_scr, v_scr):
    x = x_ref[...]
    rs = jnp.sum(x, axis=1, keepdims=True)
    rs2 = jnp.sum(x * x, axis=1, keepdims=True)
    M = M_ref[...]
    mu = jax.lax.dot_general(M, rs, (((1,), (0,)), ((), ())),
                             precision=jax.lax.Precision.HIGHEST,
                             preferred_element_type=_F32)
    ex2 = jax.lax.dot_general(M, rs2, (((1,), (0,)), ((), ())),
                              precision=jax.lax.Precision.HIGHEST,
                              preferred_element_type=_F32)
    var = ex2 - mu * mu
    gr = gw_ref[:, 0:1]
    br = gw_ref[:, 1:2]
    h_scr[...] = ((x - mu) / jnp.sqrt(var + 1e-6)) * gr + br
    bq = bias_ref[:, 0:1]
    bk = bias_ref[:, 1:2]
    bv = bias_ref[:, 2:3]
    bp = bias_ref[:, 3:4]
    for b in range(_B):
        hb = h_scr[pl.ds(b * _C, _C), :]
        q_scr[pl.ds(b * _C, _C), :] = jax.lax.dot_general(
            Wq_ref[...], hb, (((1,), (0,)), ((), ())),
            preferred_element_type=_F32) + bq
        k_scr[pl.ds(b * _C, _C), :] = jax.lax.dot_general(
            Wk_ref[...], hb, (((1,), (0,)), ((), ())),
            preferred_element_type=_F32) + bk
        v_scr[pl.ds(b * _C, _C), :] = jax.lax.dot_general(
            Wv_ref[...], hb, (((1,), (0,)), ((), ())),
            preferred_element_type=_F32) + bv
    scale = float(_C) ** (-0.5)
    bfr = lambda u: u.astype(jnp.bfloat16).astype(_F32)
    for b in range(_B):
        vb = bfr(v_scr[pl.ds(b * _C, _C), :])
        for i in range(_T):
            qi = bfr(q_scr[pl.ds(b * _C, _C), pl.ds(i * _HW, _HW)])
            rows = []
            for j in range(_T):
                kj = bfr(k_scr[pl.ds(b * _C, _C), pl.ds(j * _HW, _HW)])
                rows.append(jnp.sum(qi * kj, axis=0, keepdims=True) * scale)
            s = jnp.concatenate(rows, axis=0)
            p = bfr(jax.nn.softmax(s, axis=0))
            h2 = p[0:1, :] * vb[:, 0:_HW]
            for j in range(1, _T):
                h2 = h2 + p[j:j + 1, :] * vb[:, j * _HW:(j + 1) * _HW]
            h_scr[pl.ds(b * _C, _C), pl.ds(i * _HW, _HW)] = h2
    for b in range(_B):
        xb = x_ref[pl.ds(b * _C, _C), :]
        tdiff = jnp.concatenate(
            [jnp.zeros((_C, _HW), _F32),
             xb[:, _HW:_THW] - xb[:, 0:_THW - _HW]],
            axis=1)
        zt_ref[pl.ds(b * _C, _C), :] = jax.lax.dot_general(
            Wp_ref[...], h_scr[pl.ds(b * _C, _C), :], (((1,), (0,)), ((), ())),
            preferred_element_type=_F32) + bp + tdiff


def _attn(xT, gn_g, gn_b, Wq, bq, Wk, bk, Wv, bv, Wp, bp):
    gw = jnp.stack([jnp.tile(gn_g, (_B,)), jnp.tile(gn_b, (_B,))], axis=1)
    row = jax.lax.broadcasted_iota(jnp.int32, (_BC, _BC), 0)
    col = jax.lax.broadcasted_iota(jnp.int32, (_BC, _BC), 1)
    M = jnp.where(row // 4 == col // 4, 1.0 / (4 * _THW), 0.0).astype(_F32)
    bias = jnp.stack([bq, bk, bv, bp], axis=1)
    full = lambda a, b_: pl.BlockSpec((a, b_), lambda: (0, 0))
    return pl.pallas_call(
        _attn_body,
        grid=(),
        in_specs=[full(_BC, _THW), full(_BC, 2), full(_BC, _BC),
                  full(_C, _C), full(_C, _C), full(_C, _C), full(_C, _C),
                  full(_C, 4)],
        out_specs=full(_BC, _THW),
        out_shape=jax.ShapeDtypeStruct((_BC, _THW), _F32),
        scratch_shapes=[pltpu.VMEM((_BC, _THW), _F32),
                        pltpu.VMEM((_BC, _THW), _F32),
                        pltpu.VMEM((_BC, _THW), _F32),
                        pltpu.VMEM((_BC, _THW), _F32)],
    )(xT, gw, M, Wq, Wk, Wv, Wp, bias)


def _sc_gather_hist(emb_p, idx):
    info = plsc.get_sparse_core_info()
    nc, ns = info.num_cores, info.num_subcores
    nw = nc * ns
    bpw = _N // nw
    ksl = _K // ns
    mesh = plsc.VectorSubcoreMesh(core_axis_name="c", subcore_axis_name="s")

    @functools.partial(
        pl.kernel, mesh=mesh,
        out_type=[jax.ShapeDtypeStruct((_N, 2 * _C), _F32),
                  jax.ShapeDtypeStruct((nc * _K,), _F32)],
        scratch_types=[pltpu.VMEM((bpw,), jnp.int32),
                       pltpu.VMEM((bpw, 2 * _C), _F32),
                       pltpu.VMEM((bpw,), _F32),
                       pltpu.VMEM((ksl,), _F32),
                       pltpu.VMEM_SHARED((_K,), _F32),
                       pltpu.SemaphoreType.DMA],
    )
    def k(emb_h, idx_h, q_h, cnt_h, idx_v, rows_v, ones_v, slice_v, h_sh, sem):
        cid = jax.lax.axis_index("c")
        sid = jax.lax.axis_index("s")
        wid = sid * nc + cid
        base = wid * bpw

        def _fill(i, _):
            ones_v[pl.ds(i * 16, 16)] = jnp.ones((16,), _F32)
            return 0
        jax.lax.fori_loop(0, bpw // 16, _fill, 0)

        @pl.when(sid == 0)
        def _zero():
            def _z(i, _):
                slice_v[pl.ds(i * 16, 16)] = jnp.zeros((16,), _F32)
                return 0
            jax.lax.fori_loop(0, ksl // 16, _z, 0)
            for j in range(ns):
                pltpu.sync_copy(slice_v, h_sh.at[pl.ds(j * ksl, ksl)])
        plsc.subcore_barrier()

        pltpu.sync_copy(idx_h.at[pl.ds(base, bpw)], idx_v)
        pltpu.async_copy(emb_h.at[idx_v], rows_v, sem).wait()
        pltpu.sync_copy(rows_v, q_h.at[pl.ds(base, bpw)])
        pltpu.sync_copy(ones_v, h_sh.at[idx_v], add=True)
        plsc.subcore_barrier()

        pltpu.sync_copy(h_sh.at[pl.ds(sid * ksl, ksl)], slice_v)
        pltpu.sync_copy(slice_v, cnt_h.at[pl.ds(cid * _K + sid * ksl, ksl)])

    return k(emb_p, idx)


def _fin_body(zT_ref, qs_ref, qt_ref, avgs_ref, avgt_ref, cs_ref, ct_ref,
              zq_ref, loss_ref, l2_ref, ps_ref, pt_ref):
    acc = jnp.zeros((1, 1), _F32)
    for i in range(_N // _TB):
        rb, cb = i // 8, i % 8
        qs = qs_ref[pl.ds(i * _TB, _TB), 0:_C]
        qt = qt_ref[pl.ds(i * _TB, _TB), 0:_C]
        zq = qt + qs
        z64 = zT_ref[pl.ds(rb * _C, _C), pl.ds(cb * _HW, _HW)]
        zrows = jnp.transpose(z64, (1, 0))
        diff = zq - zrows
        acc = acc + jnp.sum(diff * diff).reshape(1, 1)
        zq_ref[pl.ds(rb * _C, _C), pl.ds(cb * _HW, _HW)] = jnp.transpose(
            zrows + diff, (1, 0))
    msq = acc * (1.0 / float(_N * _C))
    loss_ref[...] = msq + _BETA * msq

    u = jnp.full((1, _K), 1.0 / _K, _F32)
    pu = jax.nn.softmax(u, axis=1)
    lpu = jnp.log(pu)
    l2 = jnp.zeros((1, 1), _F32)
    for avg_ref in (avgs_ref, avgt_ref):
        avg = avg_ref[...] * (1.0 / float(_N))
        lsm = jax.nn.log_softmax(avg, axis=1)
        l2 = l2 + jnp.mean(pu * (lpu - lsm)).reshape(1, 1)
    l2_ref[...] = 0.1 * l2

    for c_ref, p_ref in ((cs_ref, ps_ref), (ct_ref, pt_ref)):
        cnt = c_ref[0:1, :] + c_ref[1:2, :]
        e = cnt * (1.0 / float(_N))
        p_ref[...] = jnp.exp(-jnp.sum(e * jnp.log(e + 1e-10))).reshape(1, 1)


def _finalize(zT, q_s, q_t, avg_s, avg_t, cnt_s, cnt_t):
    full = lambda a, b_: pl.BlockSpec((a, b_), lambda: (0, 0))
    one = jax.ShapeDtypeStruct((1, 1), _F32)
    return pl.pallas_call(
        _fin_body,
        grid=(),
        in_specs=[full(_BC, _THW), full(_N, 2 * _C), full(_N, 2 * _C),
                  full(1, _K), full(1, _K), full(2, _K), full(2, _K)],
        out_specs=[full(_BC, _THW), full(1, 1), full(1, 1), full(1, 1),
                   full(1, 1)],
        out_shape=[jax.ShapeDtypeStruct((_BC, _THW), _F32), one, one, one,
                   one],
    )(zT, q_s, q_t, avg_s, avg_t, cnt_s, cnt_t)


def kernel(z, emb_s, emb_t, gn_g, gn_b, Wq, bq, Wk, bk, Wv, bv, Wp, bp):
    b, c, t, h, w = z.shape
    zT = z.reshape(_BC, _THW)

    pad = jnp.zeros((_K, _C), _F32)

    idx_s2, avg_s = _vq(zT, emb_s.T)
    idx_s = idx_s2.reshape(-1)
    idx_s_out = jnp.transpose(idx_s.reshape(t, b, h, w),
                              (1, 0, 2, 3)).reshape(b * t, h * w)
    q_s, cnt_s = _sc_gather_hist(jnp.concatenate([emb_s, pad], axis=1), idx_s)

    zt_T = _attn(zT, gn_g, gn_b, Wq, bq, Wk, bk, Wv, bv, Wp, bp)
    idx_t2, avg_t = _vq(zt_T, emb_t.T)
    idx_t = idx_t2.reshape(-1)
    idx_t_out = jnp.transpose(idx_t.reshape(t, b, h, w),
                              (1, 0, 2, 3)).reshape(b * t, h * w)
    q_t, cnt_t = _sc_gather_hist(jnp.concatenate([emb_t, pad], axis=1), idx_t)

    zqT, loss, l2, perp_s, perp_t = _finalize(
        zT, q_s, q_t, avg_s, avg_t, cnt_s.reshape(2, _K), cnt_t.reshape(2, _K))

    return (zqT.reshape(z.shape), loss.reshape(()), l2.reshape(()),
            idx_s_out, idx_t_out, perp_s.reshape(()), perp_t.reshape(()))


# speedup vs baseline: 1.1470x; 1.0337x over previous
import functools

import jax
import jax.numpy as jnp
from jax.experimental import pallas as pl
from jax.experimental.pallas import tpu as pltpu
from jax.experimental.pallas import tpu_sc as plsc

_B, _C, _T, _H, _W = 2, 64, 8, 16, 16
_K = 8192
_BETA = 0.25
_NG = 16
_N = _B * _T * _H * _W
_TB = 256
_KC = 1024
_THW = _T * _H * _W
_HW = _H * _W
_BC = _B * _C
_F32 = jnp.float32


def _vq_init(embT_ref, esq_scr, avg_ref):
    for c in range(_K // _KC):
        eT = embT_ref[:, c * _KC:(c + 1) * _KC]
        esq_scr[:, c * _KC:(c + 1) * _KC] = jnp.sum(
            eT * eT, axis=0, keepdims=True)
    avg_ref[...] = jnp.zeros_like(avg_ref)


def _vq_block(z64, embT_ref, idx_ref, avg_ref, esq_scr, r_scr):
    nch = _K // _KC
    z = jnp.transpose(z64, (1, 0))
    zsq = jnp.sum(z * z, axis=1, keepdims=True)
    m = jnp.full((_TB, 1), jnp.inf, _F32)
    a = jnp.full((_TB, 1), 0.0, _F32)
    s = jnp.zeros((_TB, 1), _F32)
    for c in range(nch):
        eT = embT_ref[:, c * _KC:(c + 1) * _KC]
        esq = esq_scr[:, c * _KC:(c + 1) * _KC]
        mm = jax.lax.dot_general(z, eT, (((1,), (0,)), ((), ())),
                                 preferred_element_type=_F32)
        d = zsq + esq - 2.0 * mm
        r = 1.0 / d
        r_scr[:, c * _KC:(c + 1) * _KC] = r
        dmin = jnp.min(d, axis=1, keepdims=True)
        colf = jax.lax.broadcasted_iota(
            jnp.int32, (_TB, _KC), 1).astype(_F32) + float(c * _KC)
        idxc = jnp.min(jnp.where(d == dmin, colf, float(_K)),
                       axis=1, keepdims=True)
        upd = dmin < m
        a = jnp.where(upd, idxc, a)
        m = jnp.where(upd, dmin, m)
        s = s + jnp.sum(r, axis=1, keepdims=True)
    idx_ref[...] = a.astype(jnp.int32)
    sinvT = jnp.transpose(1.0 / s, (1, 0))
    for c in range(nch):
        r = r_scr[:, c * _KC:(c + 1) * _KC]
        avg_ref[:, c * _KC:(c + 1) * _KC] += jax.lax.dot_general(
            sinvT, r, (((1,), (0,)), ((), ())), preferred_element_type=_F32)


def _vq_body(zT_ref, embT_ref, idx_ref, avg_ref, esq_scr, r_scr):
    i = pl.program_id(0)

    @pl.when(i == 0)
    def _init():
        _vq_init(embT_ref, esq_scr, avg_ref)

    _vq_block(zT_ref[...], embT_ref, idx_ref, avg_ref, esq_scr, r_scr)


def _vq(zT, embT):
    return pl.pallas_call(
        _vq_body,
        grid=(_N // _TB,),
        in_specs=[pl.BlockSpec((_C, _HW), lambda i: (i // 8, i % 8)),
                  pl.BlockSpec((_C, _K), lambda i: (0, 0))],
        out_specs=[pl.BlockSpec((_TB, 1), lambda i: (i, 0)),
                   pl.BlockSpec((1, _K), lambda i: (0, 0))],
        out_shape=[jax.ShapeDtypeStruct((_N, 1), jnp.int32),
                   jax.ShapeDtypeStruct((1, _K), _F32)],
        scratch_shapes=[pltpu.VMEM((1, _K), _F32),
                        pltpu.VMEM((_TB, _K), _F32)],
    )(zT, embT)


def _attn_compute(x_ref, gw_ref, M_ref, Wq_ref, Wk_ref, Wv_ref, Wp_ref,
                  bias_ref, zt_scr, h_scr, q_scr, k_scr, v_scr):
    x = x_ref[...]
    rs = jnp.sum(x, axis=1, keepdims=True)
    rs2 = jnp.sum(x * x, axis=1, keepdims=True)
    M = M_ref[...]
    mu = jax.lax.dot_general(M, rs, (((1,), (0,)), ((), ())),
                             precision=jax.lax.Precision.HIGHEST,
                             preferred_element_type=_F32)
    ex2 = jax.lax.dot_general(M, rs2, (((1,), (0,)), ((), ())),
                              precision=jax.lax.Precision.HIGHEST,
                              preferred_element_type=_F32)
    var = ex2 - mu * mu
    gr = gw_ref[:, 0:1]
    br = gw_ref[:, 1:2]
    h_scr[...] = ((x - mu) / jnp.sqrt(var + 1e-6)) * gr + br
    bq = bias_ref[:, 0:1]
    bk = bias_ref[:, 1:2]
    bv = bias_ref[:, 2:3]
    bp = bias_ref[:, 3:4]
    for b in range(_B):
        hb = h_scr[pl.ds(b * _C, _C), :]
        q_scr[pl.ds(b * _C, _C), :] = jax.lax.dot_general(
            Wq_ref[...], hb, (((1,), (0,)), ((), ())),
            preferred_element_type=_F32) + bq
        k_scr[pl.ds(b * _C, _C), :] = jax.lax.dot_general(
            Wk_ref[...], hb, (((1,), (0,)), ((), ())),
            preferred_element_type=_F32) + bk
        v_scr[pl.ds(b * _C, _C), :] = jax.lax.dot_general(
            Wv_ref[...], hb, (((1,), (0,)), ((), ())),
            preferred_element_type=_F32) + bv
    scale = float(_C) ** (-0.5)
    bfr = lambda u: u.astype(jnp.bfloat16).astype(_F32)
    for b in range(_B):
        vb = bfr(v_scr[pl.ds(b * _C, _C), :])
        for i in range(_T):
            qi = bfr(q_scr[pl.ds(b * _C, _C), pl.ds(i * _HW, _HW)])
            rows = []
            for j in range(_T):
                kj = bfr(k_scr[pl.ds(b * _C, _C), pl.ds(j * _HW, _HW)])
                rows.append(jnp.sum(qi * kj, axis=0, keepdims=True) * scale)
            s = jnp.concatenate(rows, axis=0)
            p = bfr(jax.nn.softmax(s, axis=0))
            h2 = p[0:1, :] * vb[:, 0:_HW]
            for j in range(1, _T):
                h2 = h2 + p[j:j + 1, :] * vb[:, j * _HW:(j + 1) * _HW]
            h_scr[pl.ds(b * _C, _C), pl.ds(i * _HW, _HW)] = h2
    for b in range(_B):
        xb = x_ref[pl.ds(b * _C, _C), :]
        tdiff = jnp.concatenate(
            [jnp.zeros((_C, _HW), _F32),
             xb[:, _HW:_THW] - xb[:, 0:_THW - _HW]],
            axis=1)
        ztb = jax.lax.dot_general(
            Wp_ref[...], h_scr[pl.ds(b * _C, _C), :], (((1,), (0,)), ((), ())),
            preferred_element_type=_F32) + bp + tdiff
        for t in range(_T):
            zt_scr[pl.ds(b * _T + t, 1)] = ztb[:, t * _HW:(t + 1) * _HW].reshape(
                1, _C, _HW)


def _avq2_body(zT_ref, embT_ref, gw_ref, M_ref, Wq_ref, Wk_ref, Wv_ref,
               Wp_ref, bias_ref, idx_ref, avg_ref, esq_scr, r_scr, zt_scr,
               h_scr, q_scr, k_scr, v_scr):
    i = pl.program_id(0)

    @pl.when(i == 0)
    def _prologue():
        _attn_compute(zT_ref, gw_ref, M_ref, Wq_ref, Wk_ref, Wv_ref, Wp_ref,
                      bias_ref, zt_scr, h_scr, q_scr, k_scr, v_scr)
        _vq_init(embT_ref, esq_scr, avg_ref)

    @pl.when(i > 0)
    def _vq_step():
        z64 = zt_scr[pl.ds(i - 1, 1)].reshape(_C, _HW)
        _vq_block(z64, embT_ref, idx_ref, avg_ref, esq_scr, r_scr)


def _attn_vq2(xT, embT, gn_g, gn_b, Wq, bq, Wk, bk, Wv, bv, Wp, bp):
    gw = jnp.stack([jnp.tile(gn_g, (_B,)), jnp.tile(gn_b, (_B,))], axis=1)
    row = jax.lax.broadcasted_iota(jnp.int32, (_BC, _BC), 0)
    col = jax.lax.broadcasted_iota(jnp.int32, (_BC, _BC), 1)
    M = jnp.where(row // 4 == col // 4, 1.0 / (4 * _THW), 0.0).astype(_F32)
    bias = jnp.stack([bq, bk, bv, bp], axis=1)
    full = lambda a, b_: pl.BlockSpec((a, b_), lambda i: (0, 0))
    return pl.pallas_call(
        _avq2_body,
        grid=(_N // _TB + 1,),
        in_specs=[full(_BC, _THW), full(_C, _K), full(_BC, 2), full(_BC, _BC),
                  full(_C, _C), full(_C, _C), full(_C, _C), full(_C, _C),
                  full(_C, 4)],
        out_specs=[pl.BlockSpec((_TB, 1),
                                lambda i: (jnp.maximum(i - 1, 0), 0)),
                   full(1, _K)],
        out_shape=[jax.ShapeDtypeStruct((_N, 1), jnp.int32),
                   jax.ShapeDtypeStruct((1, _K), _F32)],
        scratch_shapes=[pltpu.VMEM((1, _K), _F32),
                        pltpu.VMEM((_TB, _K), _F32),
                        pltpu.VMEM((_N // _TB, _C, _HW), _F32),
                        pltpu.VMEM((_BC, _THW), _F32),
                        pltpu.VMEM((_BC, _THW), _F32),
                        pltpu.VMEM((_BC, _THW), _F32),
                        pltpu.VMEM((_BC, _THW), _F32)],
    )(xT, embT, gw, M, Wq, Wk, Wv, Wp, bias)


def _sc_gather_hist(emb_p, idx):
    info = plsc.get_sparse_core_info()
    nc, ns = info.num_cores, info.num_subcores
    nw = nc * ns
    bpw = _N // nw
    ksl = _K // ns
    mesh = plsc.VectorSubcoreMesh(core_axis_name="c", subcore_axis_name="s")

    @functools.partial(
        pl.kernel, mesh=mesh,
        out_type=[jax.ShapeDtypeStruct((_N, 2 * _C), _F32),
                  jax.ShapeDtypeStruct((nc * _K,), _F32)],
        scratch_types=[pltpu.VMEM((bpw,), jnp.int32),
                       pltpu.VMEM((bpw, 2 * _C), _F32),
                       pltpu.VMEM((bpw,), _F32),
                       pltpu.VMEM((ksl,), _F32),
                       pltpu.VMEM_SHARED((_K,), _F32),
                       pltpu.SemaphoreType.DMA],
    )
    def k(emb_h, idx_h, q_h, cnt_h, idx_v, rows_v, ones_v, slice_v, h_sh, sem):
        cid = jax.lax.axis_index("c")
        sid = jax.lax.axis_index("s")
        wid = sid * nc + cid
        base = wid * bpw

        def _fill(i, _):
            ones_v[pl.ds(i * 16, 16)] = jnp.ones((16,), _F32)
            return 0
        jax.lax.fori_loop(0, bpw // 16, _fill, 0)

        @pl.when(sid == 0)
        def _zero():
            def _z(i, _):
                slice_v[pl.ds(i * 16, 16)] = jnp.zeros((16,), _F32)
                return 0
            jax.lax.fori_loop(0, ksl // 16, _z, 0)
            for j in range(ns):
                pltpu.sync_copy(slice_v, h_sh.at[pl.ds(j * ksl, ksl)])
        plsc.subcore_barrier()

        pltpu.sync_copy(idx_h.at[pl.ds(base, bpw)], idx_v)
        pltpu.async_copy(emb_h.at[idx_v], rows_v, sem).wait()
        pltpu.sync_copy(rows_v, q_h.at[pl.ds(base, bpw)])
        pltpu.sync_copy(ones_v, h_sh.at[idx_v], add=True)
        plsc.subcore_barrier()

        pltpu.sync_copy(h_sh.at[pl.ds(sid * ksl, ksl)], slice_v)
        pltpu.sync_copy(slice_v, cnt_h.at[pl.ds(cid * _K + sid * ksl, ksl)])

    return k(emb_p, idx)


def _fin_body(zT_ref, qs_ref, qt_ref, avgs_ref, avgt_ref, cs_ref, ct_ref,
              zq_ref, loss_ref, l2_ref, ps_ref, pt_ref):
    acc = jnp.zeros((1, 1), _F32)
    for i in range(_N // _TB):
        rb, cb = i // 8, i % 8
        qs = qs_ref[pl.ds(i * _TB, _TB), 0:_C]
        qt = qt_ref[pl.ds(i * _TB, _TB), 0:_C]
        zq = qt + qs
        z64 = zT_ref[pl.ds(rb * _C, _C), pl.ds(cb * _HW, _HW)]
        zrows = jnp.transpose(z64, (1, 0))
        diff = zq - zrows
        acc = acc + jnp.sum(diff * diff).reshape(1, 1)
        zq_ref[pl.ds(rb * _C, _C), pl.ds(cb * _HW, _HW)] = jnp.transpose(
            zrows + diff, (1, 0))
    msq = acc * (1.0 / float(_N * _C))
    loss_ref[...] = msq + _BETA * msq

    u = jnp.full((1, _K), 1.0 / _K, _F32)
    pu = jax.nn.softmax(u, axis=1)
    lpu = jnp.log(pu)
    l2 = jnp.zeros((1, 1), _F32)
    for avg_ref in (avgs_ref, avgt_ref):
        avg = avg_ref[...] * (1.0 / float(_N))
        lsm = jax.nn.log_softmax(avg, axis=1)
        l2 = l2 + jnp.mean(pu * (lpu - lsm)).reshape(1, 1)
    l2_ref[...] = 0.1 * l2

    for c_ref, p_ref in ((cs_ref, ps_ref), (ct_ref, pt_ref)):
        cnt = c_ref[0:1, :] + c_ref[1:2, :]
        e = cnt * (1.0 / float(_N))
        p_ref[...] = jnp.exp(-jnp.sum(e * jnp.log(e + 1e-10))).reshape(1, 1)


def _finalize(zT, q_s, q_t, avg_s, avg_t, cnt_s, cnt_t):
    full = lambda a, b_: pl.BlockSpec((a, b_), lambda: (0, 0))
    one = jax.ShapeDtypeStruct((1, 1), _F32)
    return pl.pallas_call(
        _fin_body,
        grid=(),
        in_specs=[full(_BC, _THW), full(_N, 2 * _C), full(_N, 2 * _C),
                  full(1, _K), full(1, _K), full(2, _K), full(2, _K)],
        out_specs=[full(_BC, _THW), full(1, 1), full(1, 1), full(1, 1),
                   full(1, 1)],
        out_shape=[jax.ShapeDtypeStruct((_BC, _THW), _F32), one, one, one,
                   one],
    )(zT, q_s, q_t, avg_s, avg_t, cnt_s, cnt_t)


def kernel(z, emb_s, emb_t, gn_g, gn_b, Wq, bq, Wk, bk, Wv, bv, Wp, bp):
    b, c, t, h, w = z.shape
    zT = z.reshape(_BC, _THW)

    pad = jnp.zeros((_K, _C), _F32)

    idx_s2, avg_s = _vq(zT, emb_s.T)
    idx_s = idx_s2.reshape(-1)
    idx_s_out = jnp.transpose(idx_s.reshape(t, b, h, w),
                              (1, 0, 2, 3)).reshape(b * t, h * w)
    q_s, cnt_s = _sc_gather_hist(jnp.concatenate([emb_s, pad], axis=1), idx_s)

    idx_t2, avg_t = _attn_vq2(zT, emb_t.T, gn_g, gn_b,
                              Wq, bq, Wk, bk, Wv, bv, Wp, bp)
    idx_t = idx_t2.reshape(-1)
    idx_t_out = jnp.transpose(idx_t.reshape(t, b, h, w),
                              (1, 0, 2, 3)).reshape(b * t, h * w)
    q_t, cnt_t = _sc_gather_hist(jnp.concatenate([emb_t, pad], axis=1), idx_t)

    zqT, loss, l2, perp_s, perp_t = _finalize(
        zT, q_s, q_t, avg_s, avg_t, cnt_s.reshape(2, _K), cnt_t.reshape(2, _K))

    return (zqT.reshape(z.shape), loss.reshape(()), l2.reshape(()),
            idx_s_out, idx_t_out, perp_s.reshape(()), perp_t.reshape(()))
